# Initial kernel scaffold; baseline (speedup 1.0000x reference)
import jax
import jax.numpy as jnp
from jax.experimental import pallas as pl


def kernel(x):
    raise NotImplementedError("write your pallas kernel here")


# speedup vs baseline: 31.7622x; 31.7622x over previous
import jax
import jax.numpy as jnp
from jax import lax
from jax.experimental import pallas as pl
from jax.experimental.pallas import tpu as pltpu

CH, CW = 448, 448
TOP, LEFT = 32, 32
RB = 32
IB = 16


def _crop_body(in_ref, out_ref):
    out_ref[...] = in_ref[:, :, LEFT:LEFT + CW]


def kernel(x):
    B, T, C, H, W = x.shape
    N = B * T * C
    xf = x.reshape(N, H, W)
    out = pl.pallas_call(
        _crop_body,
        grid=(N // IB, CH // RB),
        in_specs=[pl.BlockSpec((IB, RB, W), lambda b, i: (b, i + TOP // RB, 0))],
        out_specs=pl.BlockSpec((IB, RB, CW), lambda b, i: (b, i, 0)),
        out_shape=jax.ShapeDtypeStruct((N, CH, CW), jnp.float32),
    )(xf)
    return out.reshape(B, T * C, CH, CW)
